# baseline (speedup 1.0000x reference)
import functools

import jax
import jax.numpy as jnp
from jax import lax
from jax.experimental import pallas as pl
from jax.experimental.pallas import tpu as pltpu
from jax.experimental.pallas import tpu_sc as plsc

N = 320000
D = 128
NUM_LABELS = 10000

ROW = 128
NROWS = N // ROW
ROWS_PAD = 2560
N_PAD = ROWS_PAD * ROW
CH_ROWS = ROWS_PAD // 2
CH = CH_ROWS * ROW
TC_BLK = 2560
CH0_STEPS = CH // TC_BLK
CH1_REAL = N - CH
CH1_STEPS = CH1_REAL // TC_BLK


def _tc_body(x_ref, w_ref, b_ref, s_ref, out_ref, *, tail):
    i = pl.program_id(0)

    if tail > 0:
        @pl.when(i == 0)
        def _():
            out_ref[pl.ds(CH - tail, tail)] = jnp.zeros((tail,), jnp.float32)

    h = jnp.dot(x_ref[...], w_ref[...], preferred_element_type=jnp.float32)
    r = jnp.maximum(h + b_ref[...], 0.0)
    per = lax.dot_general(s_ref[...], r, (((1,), (1,)), ((), ())),
                          preferred_element_type=jnp.float32)
    out_ref[pl.ds(i * TC_BLK, TC_BLK)] = per.reshape(TC_BLK)


def _tc_per(inputs, w_scaled, b_scaled, sgn, *, steps, blk_off, tail):
    return pl.pallas_call(
        functools.partial(_tc_body, tail=tail),
        grid=(steps,),
        in_specs=[
            pl.BlockSpec((TC_BLK, D), lambda i: (i + blk_off, 0)),
            pl.BlockSpec((D, D), lambda i: (0, 0)),
            pl.BlockSpec((1, D), lambda i: (0, 0)),
            pl.BlockSpec((1, D), lambda i: (0, 0)),
        ],
        out_specs=pl.BlockSpec((CH,), lambda i: (0,)),
        out_shape=jax.ShapeDtypeStruct((CH,), jnp.float32),
    )(inputs, w_scaled, b_scaled, sgn)


NC, NS, L = 2, 16, 16
TILE_ROWS = CH_ROWS // (NC * NS)
NL_PAD = 10240
ZCHUNK = NL_PAD // NS
SC_GRP = 10


def _sc_segment_sum(vals2d, labs2d, row_base):
    mesh = plsc.VectorSubcoreMesh(core_axis_name="c", subcore_axis_name="s")

    @functools.partial(
        pl.kernel,
        out_type=jax.ShapeDtypeStruct((NC * NL_PAD,), jnp.float32),
        mesh=mesh,
        scratch_types=[
            pltpu.VMEM((TILE_ROWS, ROW), jnp.float32),
            pltpu.VMEM((TILE_ROWS, ROW), jnp.int32),
            pltpu.VMEM((ZCHUNK,), jnp.float32),
            pltpu.VMEM_SHARED((NL_PAD,), jnp.float32),
            pltpu.SemaphoreType.DMA,
        ],
    )
    def seg_kernel(vals_hbm, labs_hbm, out_hbm, vals_v, labs_v, zero_v,
                   acc_sh, sem):
        c = lax.axis_index("c")
        s = lax.axis_index("s")

        lstart = (c * NS + s) * TILE_ROWS
        pltpu.sync_copy(vals_hbm.at[pl.ds(lstart, TILE_ROWS)], vals_v)
        pltpu.sync_copy(labs_hbm.at[pl.ds(row_base + lstart, TILE_ROWS)],
                        labs_v)

        def zbody(i, carry):
            zero_v[pl.ds(i * L, L)] = jnp.zeros((L,), jnp.float32)
            return carry

        lax.fori_loop(0, ZCHUNK // L, zbody, 0)
        pltpu.sync_copy(zero_v, acc_sh.at[pl.ds(s * ZCHUNK, ZCHUNK)])
        plsc.subcore_barrier()

        def sgroup(g, carry):
            copies = []
            for j in range(SC_GRP):
                r = g * SC_GRP + j
                copies.append(pltpu.async_copy(
                    vals_v.at[r], acc_sh.at[labs_v.at[r]], sem, add=True))
            for cp in copies:
                cp.wait()
            return carry

        lax.fori_loop(0, TILE_ROWS // SC_GRP, sgroup, 0)
        plsc.subcore_barrier()

        pltpu.sync_copy(acc_sh.at[pl.ds(s * ZCHUNK, ZCHUNK)],
                        zero_v.at[pl.ds(0, ZCHUNK)])
        pltpu.sync_copy(zero_v.at[pl.ds(0, ZCHUNK)],
                        out_hbm.at[pl.ds(c * NL_PAD + s * ZCHUNK, ZCHUNK)])

    return seg_kernel(vals2d, labs2d)


def _combine_body(p0_ref, p1_ref, out_ref):
    out_ref[...] = (p0_ref[pl.ds(0, NL_PAD)] + p0_ref[pl.ds(NL_PAD, NL_PAD)]
                    + p1_ref[pl.ds(0, NL_PAD)]
                    + p1_ref[pl.ds(NL_PAD, NL_PAD)])


def _combine(p0, p1):
    return pl.pallas_call(
        _combine_body,
        out_shape=jax.ShapeDtypeStruct((NL_PAD,), jnp.float32),
    )(p0, p1)


def kernel(inputs, labels, W_emb, b_emb, w_red):
    aw = jnp.abs(w_red)
    w_scaled = W_emb * aw[None, :]
    b_scaled = (b_emb * aw).reshape(1, D)
    sgn = jnp.where(w_red < 0, -1.0, 1.0).reshape(1, D)

    labs2d = jnp.pad(labels.astype(jnp.int32).reshape(NROWS, ROW),
                     ((0, ROWS_PAD - NROWS), (0, 0)))

    per0 = _tc_per(inputs, w_scaled, b_scaled, sgn,
                   steps=CH0_STEPS, blk_off=0, tail=0)
    per1 = _tc_per(inputs, w_scaled, b_scaled, sgn,
                   steps=CH1_STEPS, blk_off=CH0_STEPS, tail=CH - CH1_REAL)
    p0 = _sc_segment_sum(per0.reshape(CH_ROWS, ROW), labs2d, 0)
    p1 = _sc_segment_sum(per1.reshape(CH_ROWS, ROW), labs2d, CH_ROWS)
    return _combine(p0, p1)[:NUM_LABELS]


# speedup vs baseline: 1.5156x; 1.5156x over previous
import functools

import jax
import jax.numpy as jnp
from jax import lax
from jax.experimental import pallas as pl
from jax.experimental.pallas import tpu as pltpu
from jax.experimental.pallas import tpu_sc as plsc

N = 320000
D = 128
NUM_LABELS = 10000


TC_BLK = 32000
assert N % TC_BLK == 0


def _tc_body(x_ref, w_ref, b_ref, s_ref, out_ref):
    i = pl.program_id(0)

    @pl.when(i == 0)
    def _():
        out_ref[pl.ds(N, N_PAD - N)] = jnp.zeros((N_PAD - N,), jnp.float32)

    h = jnp.dot(x_ref[...], w_ref[...], preferred_element_type=jnp.float32)
    r = jnp.maximum(h + b_ref[...], 0.0)
    per = lax.dot_general(s_ref[...], r, (((1,), (1,)), ((), ())),
                          preferred_element_type=jnp.float32)
    out_ref[pl.ds(i * TC_BLK, TC_BLK)] = per.reshape(TC_BLK)


def _tc_per(inputs, W_emb, b_emb, w_red):
    grid = (N // TC_BLK,)
    aw = jnp.abs(w_red)
    w_scaled = W_emb * aw[None, :]
    b_scaled = (b_emb * aw).reshape(1, D)
    sgn = jnp.where(w_red < 0, -1.0, 1.0).reshape(1, D)
    return pl.pallas_call(
        _tc_body,
        grid=grid,
        in_specs=[
            pl.BlockSpec((TC_BLK, D), lambda i: (i, 0)),
            pl.BlockSpec((D, D), lambda i: (0, 0)),
            pl.BlockSpec((1, D), lambda i: (0, 0)),
            pl.BlockSpec((1, D), lambda i: (0, 0)),
        ],
        out_specs=pl.BlockSpec((N_PAD,), lambda i: (0,)),
        out_shape=jax.ShapeDtypeStruct((N_PAD,), jnp.float32),
    )(inputs, w_scaled, b_scaled, sgn)


NC, NS, L = 2, 16, 16
ROW = 128
NROWS = N // ROW
ROWS_PER_TILE = 160
ROWS_PAD = NS * ROWS_PER_TILE
N_PAD = ROWS_PAD * ROW
NL_PAD = 10240
OUT_CHUNK = NL_PAD // (NC * NS)
ZCHUNK = NL_PAD // NS
SC_GRP = 10


def _sc_segment_sum(vals2d, labs2d):
    mesh = plsc.VectorSubcoreMesh(core_axis_name="c", subcore_axis_name="s")

    @functools.partial(
        pl.kernel,
        out_type=jax.ShapeDtypeStruct((NL_PAD,), jnp.float32),
        mesh=mesh,
        scratch_types=[
            pltpu.VMEM((ROWS_PER_TILE, ROW), jnp.float32),
            pltpu.VMEM((ROWS_PER_TILE, ROW), jnp.int32),
            pltpu.VMEM((ZCHUNK,), jnp.float32),
            pltpu.VMEM_SHARED((NL_PAD,), jnp.float32),
            pltpu.SemaphoreType.DMA,
        ],
    )
    def seg_kernel(vals_hbm, labs_hbm, out_hbm, vals_v, labs_v,
                   zero_v, acc_sh, sem):
        c = lax.axis_index("c")
        s = lax.axis_index("s")

        base = s * ROWS_PER_TILE
        pltpu.sync_copy(vals_hbm.at[pl.ds(base, ROWS_PER_TILE)], vals_v)
        pltpu.sync_copy(labs_hbm.at[pl.ds(base, ROWS_PER_TILE)], labs_v)

        def zbody(i, carry):
            zero_v[pl.ds(i * L, L)] = jnp.zeros((L,), jnp.float32)
            return carry

        lax.fori_loop(0, ZCHUNK // L, zbody, 0)
        pltpu.sync_copy(zero_v, acc_sh.at[pl.ds(s * ZCHUNK, ZCHUNK)])
        plsc.subcore_barrier()

        def sgroup(g, carry):
            copies = []
            for j in range(SC_GRP):
                r = g * SC_GRP + j
                copies.append(pltpu.async_copy(
                    vals_v.at[r], acc_sh.at[labs_v.at[r]], sem, add=True))
            for cp in copies:
                cp.wait()
            return carry

        lax.fori_loop(0, ROWS_PER_TILE // SC_GRP, sgroup, 0)
        plsc.subcore_barrier()

        off = c * (NL_PAD // NC) + s * OUT_CHUNK
        pltpu.sync_copy(acc_sh.at[pl.ds(off, OUT_CHUNK)],
                        zero_v.at[pl.ds(0, OUT_CHUNK)])
        pltpu.sync_copy(zero_v.at[pl.ds(0, OUT_CHUNK)],
                        out_hbm.at[pl.ds(off, OUT_CHUNK)])

    return seg_kernel(vals2d, labs2d)


def kernel(inputs, labels, W_emb, b_emb, w_red):
    per = _tc_per(inputs, W_emb, b_emb, w_red)
    per2d = per.reshape(ROWS_PAD, ROW)
    labs2d = jnp.pad(labels.astype(jnp.int32).reshape(NROWS, ROW),
                     ((0, ROWS_PAD - NROWS), (0, 0)))
    out = _sc_segment_sum(per2d, labs2d)
    return out[:NUM_LABELS]
